# baseline (speedup 1.0000x reference)
import jax
import jax.numpy as jnp
from jax.experimental import pallas as pl
from jax.experimental.pallas import tpu as pltpu

C_ROWS = 64
K = 8


def _widen_mask(mask, N, D):
    rows = mask.shape[0]
    parts = []
    for t0 in range(0, N, 128):
        tw = min(128, N - t0)
        src = mask[:, t0 : t0 + tw]
        cw = tw * D
        idx = jax.lax.broadcasted_iota(jnp.int32, (rows, cw), 1) // D
        parts.append(jnp.take_along_axis(src, idx, axis=1))
    if len(parts) == 1:
        return parts[0]
    return jnp.concatenate(parts, axis=1)


def _kern(
    ids_hbm,
    emb_hbm,
    pe_ref,
    ue_hbm,
    mask_hbm,
    ids_buf,
    emb_buf,
    ue_buf,
    mask_buf,
    ids_sem,
    in_sem,
    out_sem,
    mout_sem,
):
    B = ids_hbm.shape[0]
    N = ids_hbm.shape[1]
    ND = emb_hbm.shape[1]
    D = ND // N
    NC = B // C_ROWS
    scale = float(D) ** 0.5
    pe = pe_ref[...]

    def start_in(j, slot):
        pltpu.make_async_copy(
            emb_hbm.at[pl.ds(j * C_ROWS, C_ROWS)], emb_buf.at[slot], in_sem.at[slot]
        ).start()
        pltpu.make_async_copy(
            ids_hbm.at[pl.ds(j * C_ROWS, C_ROWS)], ids_buf.at[slot], ids_sem.at[slot]
        ).start()

    for s in range(K):
        start_in(s, s)

    def body(r, carry):
        for slot in range(K):
            j = r * K + slot
            row0 = j * C_ROWS
            pltpu.make_async_copy(
                emb_hbm.at[pl.ds(row0, C_ROWS)], emb_buf.at[slot], in_sem.at[slot]
            ).wait()
            pltpu.make_async_copy(
                ids_hbm.at[pl.ds(row0, C_ROWS)], ids_buf.at[slot], ids_sem.at[slot]
            ).wait()

            @pl.when(j >= K)
            def _():
                prev0 = (j - K) * C_ROWS
                pltpu.make_async_copy(
                    ue_buf.at[slot], ue_hbm.at[pl.ds(prev0, C_ROWS)], out_sem.at[slot]
                ).wait()
                pltpu.make_async_copy(
                    mask_buf.at[slot],
                    mask_hbm.at[pl.ds(prev0, C_ROWS)],
                    mout_sem.at[slot],
                ).wait()

            mask = (ids_buf[slot] != 0).astype(jnp.float32)
            mask_buf[slot] = mask
            mask_rep = _widen_mask(mask, N, D)
            ue_buf[slot] = (emb_buf[slot] * scale + pe) * mask_rep

            pltpu.make_async_copy(
                ue_buf.at[slot], ue_hbm.at[pl.ds(row0, C_ROWS)], out_sem.at[slot]
            ).start()
            pltpu.make_async_copy(
                mask_buf.at[slot], mask_hbm.at[pl.ds(row0, C_ROWS)], mout_sem.at[slot]
            ).start()

            @pl.when(j + K < NC)
            def _():
                start_in(j + K, slot)

        return carry

    jax.lax.fori_loop(0, NC // K, body, 0)

    for s in range(K):
        j = NC - K + s
        slot = j % K
        row0 = j * C_ROWS
        pltpu.make_async_copy(
            ue_buf.at[slot], ue_hbm.at[pl.ds(row0, C_ROWS)], out_sem.at[slot]
        ).wait()
        pltpu.make_async_copy(
            mask_buf.at[slot], mask_hbm.at[pl.ds(row0, C_ROWS)], mout_sem.at[slot]
        ).wait()


def kernel(past_lengths, past_ids, past_embeddings, past_payloads, pos_emb):
    B, N = past_ids.shape
    D = past_embeddings.shape[-1]
    ND = N * D
    emb2 = past_embeddings.reshape(B, ND)
    pe2 = pos_emb.reshape(1, ND)
    ue, mask = pl.pallas_call(
        _kern,
        in_specs=[
            pl.BlockSpec(memory_space=pltpu.HBM),
            pl.BlockSpec(memory_space=pltpu.HBM),
            pl.BlockSpec(memory_space=pltpu.VMEM),
        ],
        out_specs=[
            pl.BlockSpec(memory_space=pltpu.HBM),
            pl.BlockSpec(memory_space=pltpu.HBM),
        ],
        out_shape=[
            jax.ShapeDtypeStruct((B, ND), jnp.float32),
            jax.ShapeDtypeStruct((B, N), jnp.float32),
        ],
        scratch_shapes=[
            pltpu.VMEM((K, C_ROWS, N), jnp.int32),
            pltpu.VMEM((K, C_ROWS, ND), jnp.float32),
            pltpu.VMEM((K, C_ROWS, ND), jnp.float32),
            pltpu.VMEM((K, C_ROWS, N), jnp.float32),
            pltpu.SemaphoreType.DMA((K,)),
            pltpu.SemaphoreType.DMA((K,)),
            pltpu.SemaphoreType.DMA((K,)),
            pltpu.SemaphoreType.DMA((K,)),
        ],
        compiler_params=pltpu.CompilerParams(
            vmem_limit_bytes=100 * 1024 * 1024,
        ),
    )(past_ids, emb2, pe2)
    return (past_lengths, ue.reshape(B, N, D), mask[..., None])


# speedup vs baseline: 3.3496x; 3.3496x over previous
import jax
import jax.numpy as jnp
from jax.experimental import pallas as pl
from jax.experimental.pallas import tpu as pltpu

C_N = 2
K = 10


def _kern(
    ids_hbm,
    emb_hbm,
    pe_ref,
    ue_hbm,
    mask_hbm,
    ids_buf,
    emb_buf,
    ue_buf,
    mask_buf,
    ids_sem,
    in_sem,
    out_sem,
    mout_sem,
):
    N, D, B = emb_hbm.shape
    NC = N // C_N
    scale = float(D) ** 0.5

    def start_in(j, slot):
        pltpu.make_async_copy(
            emb_hbm.at[pl.ds(j * C_N, C_N)], emb_buf.at[slot], in_sem.at[slot]
        ).start()
        pltpu.make_async_copy(
            ids_hbm.at[pl.ds(j * C_N, C_N)], ids_buf.at[slot], ids_sem.at[slot]
        ).start()

    for s in range(K):
        start_in(s, s)

    def body(r, carry):
        for slot in range(K):
            j = r * K + slot
            n0 = j * C_N
            pltpu.make_async_copy(
                emb_hbm.at[pl.ds(n0, C_N)], emb_buf.at[slot], in_sem.at[slot]
            ).wait()
            pltpu.make_async_copy(
                ids_hbm.at[pl.ds(n0, C_N)], ids_buf.at[slot], ids_sem.at[slot]
            ).wait()

            @pl.when(j >= K)
            def _():
                p0 = (j - K) * C_N
                pltpu.make_async_copy(
                    ue_buf.at[slot], ue_hbm.at[pl.ds(p0, C_N)], out_sem.at[slot]
                ).wait()
                pltpu.make_async_copy(
                    mask_buf.at[slot], mask_hbm.at[pl.ds(p0, C_N)], mout_sem.at[slot]
                ).wait()

            m = (ids_buf[slot] != 0).astype(jnp.float32)
            mask_buf[slot] = m
            pe3 = pe_ref[pl.ds(n0 * D, C_N * D), :].reshape(C_N, D, 1)
            ue_buf[slot] = (emb_buf[slot] * scale + pe3) * m[:, None, :]

            pltpu.make_async_copy(
                ue_buf.at[slot], ue_hbm.at[pl.ds(n0, C_N)], out_sem.at[slot]
            ).start()
            pltpu.make_async_copy(
                mask_buf.at[slot], mask_hbm.at[pl.ds(n0, C_N)], mout_sem.at[slot]
            ).start()

            @pl.when(j + K < NC)
            def _():
                start_in(j + K, slot)

        return carry

    jax.lax.fori_loop(0, NC // K, body, 0)

    for s in range(K):
        j = NC - K + s
        slot = j % K
        n0 = j * C_N
        pltpu.make_async_copy(
            ue_buf.at[slot], ue_hbm.at[pl.ds(n0, C_N)], out_sem.at[slot]
        ).wait()
        pltpu.make_async_copy(
            mask_buf.at[slot], mask_hbm.at[pl.ds(n0, C_N)], mout_sem.at[slot]
        ).wait()


def kernel(past_lengths, past_ids, past_embeddings, past_payloads, pos_emb):
    B, N = past_ids.shape
    D = past_embeddings.shape[-1]
    idsT = past_ids.T
    embT = jnp.transpose(past_embeddings, (1, 2, 0))
    peF = pos_emb.reshape(N * D, 1)
    ueT, maskT = pl.pallas_call(
        _kern,
        in_specs=[
            pl.BlockSpec(memory_space=pltpu.HBM),
            pl.BlockSpec(memory_space=pltpu.HBM),
            pl.BlockSpec(memory_space=pltpu.VMEM),
        ],
        out_specs=[
            pl.BlockSpec(memory_space=pltpu.HBM),
            pl.BlockSpec(memory_space=pltpu.HBM),
        ],
        out_shape=[
            jax.ShapeDtypeStruct((N, D, B), jnp.float32),
            jax.ShapeDtypeStruct((N, B), jnp.float32),
        ],
        scratch_shapes=[
            pltpu.VMEM((K, C_N, B), jnp.int32),
            pltpu.VMEM((K, C_N, D, B), jnp.float32),
            pltpu.VMEM((K, C_N, D, B), jnp.float32),
            pltpu.VMEM((K, C_N, B), jnp.float32),
            pltpu.SemaphoreType.DMA((K,)),
            pltpu.SemaphoreType.DMA((K,)),
            pltpu.SemaphoreType.DMA((K,)),
            pltpu.SemaphoreType.DMA((K,)),
        ],
        compiler_params=pltpu.CompilerParams(
            vmem_limit_bytes=100 * 1024 * 1024,
        ),
    )(idsT, embT, peF)
    ue = jnp.transpose(ueT, (2, 0, 1))
    mask = maskT.T[..., None]
    return (past_lengths, ue, mask)


# speedup vs baseline: 3.6485x; 1.0892x over previous
import jax
import jax.numpy as jnp
from jax.experimental import pallas as pl
from jax.experimental.pallas import tpu as pltpu

C_N = 2
K = 10


def _kern(
    ids_hbm,
    emb_hbm,
    pe_ref,
    ue_hbm,
    mask_hbm,
    ids_buf,
    emb_buf,
    ue_buf,
    mask_buf,
    pex_ref,
    ids_sem,
    in_sem,
    out_sem,
    mout_sem,
):
    N, D, B = emb_hbm.shape
    NC = N // C_N
    scale = float(D) ** 0.5

    def start_in(j, slot):
        pltpu.make_async_copy(
            emb_hbm.at[pl.ds(j * C_N, C_N)], emb_buf.at[slot], in_sem.at[slot]
        ).start()
        pltpu.make_async_copy(
            ids_hbm.at[pl.ds(j * C_N, C_N)], ids_buf.at[slot], ids_sem.at[slot]
        ).start()

    for s in range(K):
        start_in(s, s)

    pex_ref[...] = jnp.transpose(pe_ref[...], (1, 0))[:, :, None]

    def body(r, carry):
        for slot in range(K):
            j = r * K + slot
            n0 = j * C_N
            pltpu.make_async_copy(
                emb_hbm.at[pl.ds(n0, C_N)], emb_buf.at[slot], in_sem.at[slot]
            ).wait()
            pltpu.make_async_copy(
                ids_hbm.at[pl.ds(n0, C_N)], ids_buf.at[slot], ids_sem.at[slot]
            ).wait()

            @pl.when(j >= K)
            def _():
                p0 = (j - K) * C_N
                pltpu.make_async_copy(
                    ue_buf.at[slot], ue_hbm.at[pl.ds(p0, C_N)], out_sem.at[slot]
                ).wait()
                pltpu.make_async_copy(
                    mask_buf.at[slot], mask_hbm.at[pl.ds(p0, C_N)], mout_sem.at[slot]
                ).wait()

            m = (ids_buf[slot] != 0).astype(jnp.float32)
            mask_buf[slot] = m
            pe3 = pex_ref[pl.ds(n0, C_N)]
            ue_buf[slot] = (emb_buf[slot] * scale + pe3) * m[:, None, :]

            pltpu.make_async_copy(
                ue_buf.at[slot], ue_hbm.at[pl.ds(n0, C_N)], out_sem.at[slot]
            ).start()
            pltpu.make_async_copy(
                mask_buf.at[slot], mask_hbm.at[pl.ds(n0, C_N)], mout_sem.at[slot]
            ).start()

            @pl.when(j + K < NC)
            def _():
                start_in(j + K, slot)

        return carry

    jax.lax.fori_loop(0, NC // K, body, 0)

    for s in range(K):
        j = NC - K + s
        slot = j % K
        n0 = j * C_N
        pltpu.make_async_copy(
            ue_buf.at[slot], ue_hbm.at[pl.ds(n0, C_N)], out_sem.at[slot]
        ).wait()
        pltpu.make_async_copy(
            mask_buf.at[slot], mask_hbm.at[pl.ds(n0, C_N)], mout_sem.at[slot]
        ).wait()


def kernel(past_lengths, past_ids, past_embeddings, past_payloads, pos_emb):
    B, N = past_ids.shape
    D = past_embeddings.shape[-1]
    idsT = past_ids.T
    embT = jnp.transpose(past_embeddings, (1, 2, 0))
    peT = pos_emb.T
    ueT, maskT = pl.pallas_call(
        _kern,
        in_specs=[
            pl.BlockSpec(memory_space=pltpu.HBM),
            pl.BlockSpec(memory_space=pltpu.HBM),
            pl.BlockSpec(memory_space=pltpu.VMEM),
        ],
        out_specs=[
            pl.BlockSpec(memory_space=pltpu.HBM),
            pl.BlockSpec(memory_space=pltpu.HBM),
        ],
        out_shape=[
            jax.ShapeDtypeStruct((N, D, B), jnp.float32),
            jax.ShapeDtypeStruct((N, B), jnp.float32),
        ],
        scratch_shapes=[
            pltpu.VMEM((K, C_N, B), jnp.int32),
            pltpu.VMEM((K, C_N, D, B), jnp.float32),
            pltpu.VMEM((K, C_N, D, B), jnp.float32),
            pltpu.VMEM((K, C_N, B), jnp.float32),
            pltpu.VMEM((N, D, 1), jnp.float32),
            pltpu.SemaphoreType.DMA((K,)),
            pltpu.SemaphoreType.DMA((K,)),
            pltpu.SemaphoreType.DMA((K,)),
            pltpu.SemaphoreType.DMA((K,)),
        ],
        compiler_params=pltpu.CompilerParams(
            vmem_limit_bytes=100 * 1024 * 1024,
        ),
    )(idsT, embT, peT)
    ue = jnp.transpose(ueT, (2, 0, 1))
    mask = maskT.T[..., None]
    return (past_lengths, ue, mask)


# speedup vs baseline: 3.6487x; 1.0000x over previous
import jax
import jax.numpy as jnp
from jax.experimental import pallas as pl
from jax.experimental.pallas import tpu as pltpu

C_N = 4
K = 5


def _kern(
    ids_hbm,
    emb_hbm,
    pe_ref,
    ue_hbm,
    mask_hbm,
    ids_buf,
    emb_buf,
    ue_buf,
    mask_buf,
    pex_ref,
    ids_sem,
    in_sem,
    out_sem,
    mout_sem,
):
    N, D, B = emb_hbm.shape
    NC = N // C_N
    scale = float(D) ** 0.5

    def start_in(j, slot):
        pltpu.make_async_copy(
            emb_hbm.at[pl.ds(j * C_N, C_N)], emb_buf.at[slot], in_sem.at[slot]
        ).start()
        pltpu.make_async_copy(
            ids_hbm.at[pl.ds(j * C_N, C_N)], ids_buf.at[slot], ids_sem.at[slot]
        ).start()

    for s in range(K):
        start_in(s, s)

    pex_ref[...] = jnp.transpose(pe_ref[...], (1, 0))[:, :, None]

    def body(r, carry):
        for slot in range(K):
            j = r * K + slot
            n0 = j * C_N
            pltpu.make_async_copy(
                emb_hbm.at[pl.ds(n0, C_N)], emb_buf.at[slot], in_sem.at[slot]
            ).wait()
            pltpu.make_async_copy(
                ids_hbm.at[pl.ds(n0, C_N)], ids_buf.at[slot], ids_sem.at[slot]
            ).wait()

            @pl.when(j >= K)
            def _():
                p0 = (j - K) * C_N
                pltpu.make_async_copy(
                    ue_buf.at[slot], ue_hbm.at[pl.ds(p0, C_N)], out_sem.at[slot]
                ).wait()
                pltpu.make_async_copy(
                    mask_buf.at[slot], mask_hbm.at[pl.ds(p0, C_N)], mout_sem.at[slot]
                ).wait()

            m = (ids_buf[slot] != 0).astype(jnp.float32)
            mask_buf[slot] = m
            pe3 = pex_ref[pl.ds(n0, C_N)]
            ue_buf[slot] = (emb_buf[slot] * scale + pe3) * m[:, None, :]

            pltpu.make_async_copy(
                ue_buf.at[slot], ue_hbm.at[pl.ds(n0, C_N)], out_sem.at[slot]
            ).start()
            pltpu.make_async_copy(
                mask_buf.at[slot], mask_hbm.at[pl.ds(n0, C_N)], mout_sem.at[slot]
            ).start()

            @pl.when(j + K < NC)
            def _():
                start_in(j + K, slot)

        return carry

    jax.lax.fori_loop(0, NC // K, body, 0)

    for s in range(K):
        j = NC - K + s
        slot = j % K
        n0 = j * C_N
        pltpu.make_async_copy(
            ue_buf.at[slot], ue_hbm.at[pl.ds(n0, C_N)], out_sem.at[slot]
        ).wait()
        pltpu.make_async_copy(
            mask_buf.at[slot], mask_hbm.at[pl.ds(n0, C_N)], mout_sem.at[slot]
        ).wait()


def kernel(past_lengths, past_ids, past_embeddings, past_payloads, pos_emb):
    B, N = past_ids.shape
    D = past_embeddings.shape[-1]
    idsT = past_ids.T
    embT = jnp.transpose(past_embeddings, (1, 2, 0))
    peT = pos_emb.T
    ueT, maskT = pl.pallas_call(
        _kern,
        in_specs=[
            pl.BlockSpec(memory_space=pltpu.HBM),
            pl.BlockSpec(memory_space=pltpu.HBM),
            pl.BlockSpec(memory_space=pltpu.VMEM),
        ],
        out_specs=[
            pl.BlockSpec(memory_space=pltpu.HBM),
            pl.BlockSpec(memory_space=pltpu.HBM),
        ],
        out_shape=[
            jax.ShapeDtypeStruct((N, D, B), jnp.float32),
            jax.ShapeDtypeStruct((N, B), jnp.float32),
        ],
        scratch_shapes=[
            pltpu.VMEM((K, C_N, B), jnp.int32),
            pltpu.VMEM((K, C_N, D, B), jnp.float32),
            pltpu.VMEM((K, C_N, D, B), jnp.float32),
            pltpu.VMEM((K, C_N, B), jnp.float32),
            pltpu.VMEM((N, D, 1), jnp.float32),
            pltpu.SemaphoreType.DMA((K,)),
            pltpu.SemaphoreType.DMA((K,)),
            pltpu.SemaphoreType.DMA((K,)),
            pltpu.SemaphoreType.DMA((K,)),
        ],
        compiler_params=pltpu.CompilerParams(
            vmem_limit_bytes=100 * 1024 * 1024,
        ),
    )(idsT, embT, peT)
    ue = jnp.transpose(ueT, (2, 0, 1))
    mask = maskT.T[..., None]
    return (past_lengths, ue, mask)
